# Initial kernel scaffold; baseline (speedup 1.0000x reference)
import jax
import jax.numpy as jnp
from jax.experimental import pallas as pl


def kernel(rays_o, rays_d, data):
    raise NotImplementedError("write your pallas kernel here")


# speedup vs baseline: 1.8571x; 1.8571x over previous
import functools
import math

import jax
import jax.numpy as jnp
from jax import lax
from jax.experimental import pallas as pl
from jax.experimental.pallas import tpu as pltpu
from jax.experimental.pallas import tpu_sc as plsc

RES = 128
RADIUS = 1.3
SH_DEG = 2
N_SH = (SH_DEG + 1) ** 2
CH = N_SH * 3 + 1
PADC = 32
STEP = (RADIUS * 2.0 / RES) / 2.0
N_INT = int(math.ceil(math.sqrt(3.0) * 2 * RES))
NS = N_INT - 1
NPAD = 512
RB = 64
NBLK = 1024 // RB
NPTS = 1024 * NPAD

SC_CORES = 2
SC_SUBCORES = 16
NW = SC_CORES * SC_SUBCORES
ROWS_PER_WORKER = RB // 2
ROWS_PER_CHUNK = 4
NCHUNK = ROWS_PER_WORKER // ROWS_PER_CHUNK
QT = NPAD // 128

C0 = 0.28209479177387814
C1 = 0.4886025119029199
C2 = (1.0925484305920792, -1.0925484305920792, 0.31539156525252005,
      -1.0925484305920792, 0.5462742152960396)


def _ray_start(o, d):
    opos = (RADIUS - o) / d
    oneg = (-RADIUS - o) / d
    return jnp.max(jnp.minimum(opos, oneg), axis=-1, keepdims=True)


def _idxw_body(o_ref, d_ref, idx_ref, w_ref):
    o = o_ref[...]
    d = d_ref[...]
    start = _ray_start(o, d)
    nf = lax.broadcasted_iota(jnp.int32, (RB, NPAD), 1).astype(jnp.float32)
    t = start + nf * STEP
    cs, fs, ws = [], [], []
    for a in range(3):
        pa = o[:, a:a + 1] + t * d[:, a:a + 1]
        ca = jnp.clip((pa / RADIUS + 1.0) * 0.5 * (RES - 1), 0.0, RES - 1)
        fa = jnp.floor(ca)
        cs.append(ca)
        fs.append(fa)
        ws.append(ca - fa)
    x0 = jnp.clip(fs[0].astype(jnp.int32), 0, RES - 1)
    y0 = jnp.clip(fs[1].astype(jnp.int32), 0, RES - 1)
    z0 = jnp.clip(fs[2].astype(jnp.int32), 0, RES - 1)
    x1 = jnp.minimum(x0 + 1, RES - 1)
    y1 = jnp.minimum(y0 + 1, RES - 1)
    z1 = jnp.minimum(z0 + 1, RES - 1)
    wx, wy, wz = ws
    for k in range(8):
        dz, dy, dx = (k >> 2) & 1, (k >> 1) & 1, k & 1
        zi = z1 if dz else z0
        yi = y1 if dy else y0
        xi = x1 if dx else x0
        flat = (zi * RES + yi) * RES + xi
        idx_ref[0, k] = jnp.clip(flat, 0, RES ** 3 - 1)
        wk = ((wz if dz else 1.0 - wz) * (wy if dy else 1.0 - wy)
              * (wx if dx else 1.0 - wx))
        w_ref[0, k] = wk


def _idxw_call(rays_o, rays_d):
    return pl.pallas_call(
        _idxw_body,
        grid=(NBLK,),
        in_specs=[
            pl.BlockSpec((RB, 3), lambda i: (i, 0)),
            pl.BlockSpec((RB, 3), lambda i: (i, 0)),
        ],
        out_specs=[
            pl.BlockSpec((1, 8, RB, NPAD), lambda i: (i, 0, 0, 0)),
            pl.BlockSpec((1, 8, RB, NPAD), lambda i: (i, 0, 0, 0)),
        ],
        out_shape=[
            jax.ShapeDtypeStruct((NBLK, 8, RB, NPAD), jnp.int32),
            jax.ShapeDtypeStruct((NBLK, 8, RB, NPAD), jnp.float32),
        ],
    )(rays_o, rays_d)


def _sc_gather_body(table_hbm, idx_hbm, w_hbm, out_hbm, idxb, wb, rows, acc,
                    sem):
    wid = lax.axis_index("s") * SC_CORES + lax.axis_index("c")
    rb = wid // 2
    half = wid % 2

    def chunk_body(c, _):
        row0 = half * ROWS_PER_WORKER + c * ROWS_PER_CHUNK
        pltpu.sync_copy(idx_hbm.at[rb, :, pl.ds(row0, ROWS_PER_CHUNK), :],
                        idxb)
        pltpu.sync_copy(w_hbm.at[rb, :, pl.ds(row0, ROWS_PER_CHUNK), :], wb)

        def tile_body(tt, _):
            r = tt // QT
            q = tt % QT
            copies = [
                pltpu.async_copy(
                    table_hbm.at[idxb.at[k, r, pl.ds(q * 128, 128)]],
                    rows.at[k], sem)
                for k in range(8)
            ]
            for cp in copies:
                cp.wait()

            def group_body(g, _):
                j0 = g * 16
                wvs = [wb[k, r, pl.ds(q * 128 + j0, 16)] for k in range(8)]
                for jj in range(16):
                    j = j0 + jj
                    s0 = jnp.zeros((16,), jnp.float32)
                    s1 = jnp.zeros((16,), jnp.float32)
                    for k in range(8):
                        wk = wvs[k][jj]
                        s0 = s0 + rows[k, j, 0:16] * wk
                        s1 = s1 + rows[k, j, 16:32] * wk
                    acc[j, 0:16] = s0
                    acc[j, 16:32] = s1
                return 0

            lax.fori_loop(0, 8, group_body, 0)
            base = (rb * RB + row0 + r) * NPAD + q * 128
            pltpu.sync_copy(acc, out_hbm.at[pl.ds(base, 128), :])
            return 0

        lax.fori_loop(0, ROWS_PER_CHUNK * QT, tile_body, 0)
        return 0

    lax.fori_loop(0, NCHUNK, chunk_body, 0)


@functools.lru_cache(maxsize=1)
def _sc_gather_kernel():
    return pl.kernel(
        _sc_gather_body,
        out_type=jax.ShapeDtypeStruct((NPTS, PADC), jnp.float32),
        mesh=plsc.VectorSubcoreMesh(core_axis_name="c", subcore_axis_name="s",
                                    num_cores=SC_CORES,
                                    num_subcores=SC_SUBCORES),
        scratch_types=[
            pltpu.VMEM((8, ROWS_PER_CHUNK, NPAD), jnp.int32),
            pltpu.VMEM((8, ROWS_PER_CHUNK, NPAD), jnp.float32),
            pltpu.VMEM((8, 128, PADC), jnp.float32),
            pltpu.VMEM((128, PADC), jnp.float32),
            pltpu.SemaphoreType.DMA,
        ],
        compiler_params=pltpu.CompilerParams(use_tc_tiling_on_sc=False),
    )


CRB = 32


def _compose_body(o_ref, d_ref, interp_ref, out_ref):
    o = o_ref[...]
    d = d_ref[...]
    interp = interp_ref[0]
    start = _ray_start(o, d)
    dn = jnp.sqrt(jnp.sum(d * d, axis=-1, keepdims=True))
    dist = STEP * dn
    ni = lax.broadcasted_iota(jnp.int32, (CRB, NPAD), 1)
    t = start + ni.astype(jnp.float32) * STEP
    mask = ni < NS
    for a in range(3):
        pa = o[:, a:a + 1] + t * d[:, a:a + 1]
        mask = mask & (pa > -RADIUS) & (pa < RADIUS)
    x = d[:, 0:1]
    y = d[:, 1:2]
    z = d[:, 2:3]
    sh = jnp.concatenate([
        jnp.full_like(x, C0),
        -C1 * y, C1 * z, -C1 * x,
        C2[0] * x * y, C2[1] * y * z,
        C2[2] * (2.0 * z * z - x * x - y * y),
        C2[3] * x * z, C2[4] * (x * x - y * y),
    ], axis=-1)
    shmext = jnp.concatenate(
        [sh, sh, sh, jnp.ones((CRB, 1), jnp.float32),
         jnp.zeros((CRB, PADC - CH), jnp.float32)], axis=-1)
    u = (interp * shmext[:, None, :]).reshape(CRB * NPAD, PADC)
    chi = lax.broadcasted_iota(jnp.int32, (PADC, 8), 0)
    cci = lax.broadcasted_iota(jnp.int32, (PADC, 8), 1)
    sel = (((chi < 27) & (chi // 9 == cci)) | ((chi == 27) & (cci == 3)))
    out4 = jnp.dot(u, sel.astype(jnp.float32),
                   preferred_element_type=jnp.float32)
    out4 = out4.reshape(CRB, NPAD, 8)
    sigma = jnp.maximum(jnp.where(mask, out4[:, :, 3], 0.0), 0.0)
    sigd = sigma * dist
    alpha = 1.0 - jnp.exp(-sigd)
    mi = lax.broadcasted_iota(jnp.int32, (NPAD, NPAD), 0)
    nj = lax.broadcasted_iota(jnp.int32, (NPAD, NPAD), 1)
    lower = (mi < nj).astype(jnp.float32)
    cum = jnp.dot(sigd, lower, preferred_element_type=jnp.float32)
    trans = jnp.exp(-cum)
    wlight = alpha * trans
    bg = 1.0 - jnp.sum(wlight, axis=-1)
    cols = []
    for c in range(3):
        rgb_c = jax.nn.sigmoid(jnp.where(mask, out4[:, :, c], 0.0))
        cols.append(jnp.sum(wlight * rgb_c, axis=-1) + bg)
    out_ref[...] = jnp.stack(cols, axis=-1)


def _compose_call(rays_o, rays_d, interp4):
    return pl.pallas_call(
        _compose_body,
        grid=(1024 // CRB,),
        in_specs=[
            pl.BlockSpec((CRB, 3), lambda i: (i, 0)),
            pl.BlockSpec((CRB, 3), lambda i: (i, 0)),
            pl.BlockSpec((1, CRB, NPAD, PADC), lambda i: (i, 0, 0, 0)),
        ],
        out_specs=pl.BlockSpec((CRB, 3), lambda i: (i, 0)),
        out_shape=jax.ShapeDtypeStruct((1024, 3), jnp.float32),
    )(rays_o, rays_d, interp4)


def kernel(rays_o, rays_d, data):
    table = jnp.pad(data.reshape(CH, RES ** 3).T,
                    ((0, 0), (0, PADC - CH)))
    idx, w = _idxw_call(rays_o, rays_d)
    interp = _sc_gather_kernel()(table, idx, w)
    interp4 = interp.reshape(1024 // CRB, CRB, NPAD, PADC)
    return _compose_call(rays_o, rays_d, interp4)


# speedup vs baseline: 1.9827x; 1.0676x over previous
import functools
import math

import jax
import jax.numpy as jnp
from jax import lax
from jax.experimental import pallas as pl
from jax.experimental.pallas import tpu as pltpu
from jax.experimental.pallas import tpu_sc as plsc

RES = 128
RADIUS = 1.3
SH_DEG = 2
N_SH = (SH_DEG + 1) ** 2
CH = N_SH * 3 + 1
PADC = 32
STEP = (RADIUS * 2.0 / RES) / 2.0
N_INT = int(math.ceil(math.sqrt(3.0) * 2 * RES))
NS = N_INT - 1
NPAD = 512
RB = 64
NBLK = 1024 // RB
NPTS = 1024 * NPAD

SC_CORES = 2
SC_SUBCORES = 16
NW = SC_CORES * SC_SUBCORES
ROWS_PER_WORKER = RB // 2
ROWS_PER_CHUNK = 4
NCHUNK = ROWS_PER_WORKER // ROWS_PER_CHUNK
QT = NPAD // 128

C0 = 0.28209479177387814
C1 = 0.4886025119029199
C2 = (1.0925484305920792, -1.0925484305920792, 0.31539156525252005,
      -1.0925484305920792, 0.5462742152960396)


def _ray_start(o, d):
    opos = (RADIUS - o) / d
    oneg = (-RADIUS - o) / d
    return jnp.max(jnp.minimum(opos, oneg), axis=-1, keepdims=True)


def _idxw_body(o_ref, d_ref, idx_ref, w_ref):
    o = o_ref[...]
    d = d_ref[...]
    start = _ray_start(o, d)
    nf = lax.broadcasted_iota(jnp.int32, (RB, NPAD), 1).astype(jnp.float32)
    t = start + nf * STEP
    cs, fs, ws = [], [], []
    for a in range(3):
        pa = o[:, a:a + 1] + t * d[:, a:a + 1]
        ca = jnp.clip((pa / RADIUS + 1.0) * 0.5 * (RES - 1), 0.0, RES - 1)
        fa = jnp.floor(ca)
        cs.append(ca)
        fs.append(fa)
        ws.append(ca - fa)
    x0 = jnp.clip(fs[0].astype(jnp.int32), 0, RES - 1)
    y0 = jnp.clip(fs[1].astype(jnp.int32), 0, RES - 1)
    z0 = jnp.clip(fs[2].astype(jnp.int32), 0, RES - 1)
    x1 = jnp.minimum(x0 + 1, RES - 1)
    y1 = jnp.minimum(y0 + 1, RES - 1)
    z1 = jnp.minimum(z0 + 1, RES - 1)
    wx, wy, wz = ws
    for k in range(8):
        dz, dy, dx = (k >> 2) & 1, (k >> 1) & 1, k & 1
        zi = z1 if dz else z0
        yi = y1 if dy else y0
        xi = x1 if dx else x0
        flat = (zi * RES + yi) * RES + xi
        idx_ref[0, k] = jnp.clip(flat, 0, RES ** 3 - 1)
        wk = ((wz if dz else 1.0 - wz) * (wy if dy else 1.0 - wy)
              * (wx if dx else 1.0 - wx))
        w_ref[0, k] = wk


def _idxw_call(rays_o, rays_d):
    return pl.pallas_call(
        _idxw_body,
        grid=(NBLK,),
        in_specs=[
            pl.BlockSpec((RB, 3), lambda i: (i, 0)),
            pl.BlockSpec((RB, 3), lambda i: (i, 0)),
        ],
        out_specs=[
            pl.BlockSpec((1, 8, RB, NPAD), lambda i: (i, 0, 0, 0)),
            pl.BlockSpec((1, 8, RB, NPAD), lambda i: (i, 0, 0, 0)),
        ],
        out_shape=[
            jax.ShapeDtypeStruct((NBLK, 8, RB, NPAD), jnp.int32),
            jax.ShapeDtypeStruct((NBLK, 8, RB, NPAD), jnp.float32),
        ],
    )(rays_o, rays_d)


def _sc_gather_body(table_hbm, idx_hbm, w_hbm, out_hbm, idxb, wb, rows, acc,
                    sem_a, sem_b, sem_o):
    wid = lax.axis_index("s") * SC_CORES + lax.axis_index("c")
    rb = wid // 2
    half = wid % 2
    sems = (sem_a, sem_b)
    ntiles = ROWS_PER_CHUNK * QT

    def fire(tt, buf):
        r = tt // QT
        q = tt % QT
        return [
            pltpu.async_copy(
                table_hbm.at[idxb.at[k, r, pl.ds(q * 128, 128)]],
                rows.at[buf, k], sems[buf])
            for k in range(8)
        ]

    def chunk_body(c, _):
        row0 = half * ROWS_PER_WORKER + c * ROWS_PER_CHUNK
        pltpu.sync_copy(idx_hbm.at[rb, :, pl.ds(row0, ROWS_PER_CHUNK), :],
                        idxb)
        pltpu.sync_copy(w_hbm.at[rb, :, pl.ds(row0, ROWS_PER_CHUNK), :], wb)
        out_handles = []
        pending = fire(0, 0)
        for tt in range(ntiles):
            buf = tt % 2
            nxt = fire(tt + 1, 1 - buf) if tt + 1 < ntiles else None
            for cp in pending:
                cp.wait()
            if len(out_handles) >= 2:
                out_handles.pop(0).wait()
            r = tt // QT
            q = tt % QT

            def group_body(g, _, buf=buf, r=r, q=q):
                j0 = g * 16
                wvs = [wb[k, r, pl.ds(q * 128 + j0, 16)] for k in range(8)]
                for jj in range(16):
                    j = j0 + jj
                    s0 = jnp.zeros((16,), jnp.float32)
                    s1 = jnp.zeros((16,), jnp.float32)
                    for k in range(8):
                        wk = wvs[k][jj]
                        s0 = s0 + rows[buf, k, j, 0:16] * wk
                        s1 = s1 + rows[buf, k, j, 16:32] * wk
                    acc[buf, j, 0:16] = s0
                    acc[buf, j, 16:32] = s1
                return 0

            lax.fori_loop(0, 8, group_body, 0)
            base = (rb * RB + row0 + r) * NPAD + q * 128
            out_handles.append(
                pltpu.async_copy(acc.at[buf],
                                 out_hbm.at[pl.ds(base, 128), :], sem_o))
            if nxt is not None:
                pending = nxt
        for h in out_handles:
            h.wait()
        return 0

    lax.fori_loop(0, NCHUNK, chunk_body, 0)


@functools.lru_cache(maxsize=1)
def _sc_gather_kernel():
    return pl.kernel(
        _sc_gather_body,
        out_type=jax.ShapeDtypeStruct((NPTS, PADC), jnp.float32),
        mesh=plsc.VectorSubcoreMesh(core_axis_name="c", subcore_axis_name="s",
                                    num_cores=SC_CORES,
                                    num_subcores=SC_SUBCORES),
        scratch_types=[
            pltpu.VMEM((8, ROWS_PER_CHUNK, NPAD), jnp.int32),
            pltpu.VMEM((8, ROWS_PER_CHUNK, NPAD), jnp.float32),
            pltpu.VMEM((2, 8, 128, PADC), jnp.float32),
            pltpu.VMEM((2, 128, PADC), jnp.float32),
            pltpu.SemaphoreType.DMA,
            pltpu.SemaphoreType.DMA,
            pltpu.SemaphoreType.DMA,
        ],
        compiler_params=pltpu.CompilerParams(use_tc_tiling_on_sc=False),
    )


CRB = 32


def _compose_body(o_ref, d_ref, interp_ref, out_ref):
    o = o_ref[...]
    d = d_ref[...]
    interp = interp_ref[0]
    start = _ray_start(o, d)
    dn = jnp.sqrt(jnp.sum(d * d, axis=-1, keepdims=True))
    dist = STEP * dn
    ni = lax.broadcasted_iota(jnp.int32, (CRB, NPAD), 1)
    t = start + ni.astype(jnp.float32) * STEP
    mask = ni < NS
    for a in range(3):
        pa = o[:, a:a + 1] + t * d[:, a:a + 1]
        mask = mask & (pa > -RADIUS) & (pa < RADIUS)
    x = d[:, 0:1]
    y = d[:, 1:2]
    z = d[:, 2:3]
    sh = jnp.concatenate([
        jnp.full_like(x, C0),
        -C1 * y, C1 * z, -C1 * x,
        C2[0] * x * y, C2[1] * y * z,
        C2[2] * (2.0 * z * z - x * x - y * y),
        C2[3] * x * z, C2[4] * (x * x - y * y),
    ], axis=-1)
    shmext = jnp.concatenate(
        [sh, sh, sh, jnp.ones((CRB, 1), jnp.float32),
         jnp.zeros((CRB, PADC - CH), jnp.float32)], axis=-1)
    u = (interp * shmext[:, None, :]).reshape(CRB * NPAD, PADC)
    chi = lax.broadcasted_iota(jnp.int32, (PADC, 8), 0)
    cci = lax.broadcasted_iota(jnp.int32, (PADC, 8), 1)
    sel = (((chi < 27) & (chi // 9 == cci)) | ((chi == 27) & (cci == 3)))
    out4 = jnp.dot(u, sel.astype(jnp.float32),
                   preferred_element_type=jnp.float32)
    out4 = out4.reshape(CRB, NPAD, 8)
    sigma = jnp.maximum(jnp.where(mask, out4[:, :, 3], 0.0), 0.0)
    sigd = sigma * dist
    alpha = 1.0 - jnp.exp(-sigd)
    mi = lax.broadcasted_iota(jnp.int32, (NPAD, NPAD), 0)
    nj = lax.broadcasted_iota(jnp.int32, (NPAD, NPAD), 1)
    lower = (mi < nj).astype(jnp.float32)
    cum = jnp.dot(sigd, lower, preferred_element_type=jnp.float32)
    trans = jnp.exp(-cum)
    wlight = alpha * trans
    bg = 1.0 - jnp.sum(wlight, axis=-1)
    cols = []
    for c in range(3):
        rgb_c = jax.nn.sigmoid(jnp.where(mask, out4[:, :, c], 0.0))
        cols.append(jnp.sum(wlight * rgb_c, axis=-1) + bg)
    out_ref[...] = jnp.stack(cols, axis=-1)


def _compose_call(rays_o, rays_d, interp4):
    return pl.pallas_call(
        _compose_body,
        grid=(1024 // CRB,),
        in_specs=[
            pl.BlockSpec((CRB, 3), lambda i: (i, 0)),
            pl.BlockSpec((CRB, 3), lambda i: (i, 0)),
            pl.BlockSpec((1, CRB, NPAD, PADC), lambda i: (i, 0, 0, 0)),
        ],
        out_specs=pl.BlockSpec((CRB, 3), lambda i: (i, 0)),
        out_shape=jax.ShapeDtypeStruct((1024, 3), jnp.float32),
    )(rays_o, rays_d, interp4)


def kernel(rays_o, rays_d, data):
    table = jnp.pad(data.reshape(CH, RES ** 3).T,
                    ((0, 0), (0, PADC - CH)))
    idx, w = _idxw_call(rays_o, rays_d)
    interp = _sc_gather_kernel()(table, idx, w)
    interp4 = interp.reshape(1024 // CRB, CRB, NPAD, PADC)
    return _compose_call(rays_o, rays_d, interp4)


# speedup vs baseline: 1.9867x; 1.0021x over previous
import functools
import math

import jax
import jax.numpy as jnp
from jax import lax
from jax.experimental import pallas as pl
from jax.experimental.pallas import tpu as pltpu
from jax.experimental.pallas import tpu_sc as plsc

RES = 128
RADIUS = 1.3
SH_DEG = 2
N_SH = (SH_DEG + 1) ** 2
CH = N_SH * 3 + 1
PADC = 32
STEP = (RADIUS * 2.0 / RES) / 2.0
N_INT = int(math.ceil(math.sqrt(3.0) * 2 * RES))
NS = N_INT - 1
NPAD = 512
RB = 64
NBLK = 1024 // RB
NPTS = 1024 * NPAD

SC_CORES = 2
SC_SUBCORES = 16
NW = SC_CORES * SC_SUBCORES
ROWS_PER_WORKER = RB // 2
ROWS_PER_CHUNK = 4
NCHUNK = ROWS_PER_WORKER // ROWS_PER_CHUNK
QT = NPAD // 128

C0 = 0.28209479177387814
C1 = 0.4886025119029199
C2 = (1.0925484305920792, -1.0925484305920792, 0.31539156525252005,
      -1.0925484305920792, 0.5462742152960396)


def _ray_start(o, d):
    opos = (RADIUS - o) / d
    oneg = (-RADIUS - o) / d
    return jnp.max(jnp.minimum(opos, oneg), axis=-1, keepdims=True)


def _idxw_body(o_ref, d_ref, idx_ref, w_ref):
    o = o_ref[...]
    d = d_ref[...]
    start = _ray_start(o, d)
    nf = lax.broadcasted_iota(jnp.int32, (RB, NPAD), 1).astype(jnp.float32)
    t = start + nf * STEP
    cs, fs, ws = [], [], []
    for a in range(3):
        pa = o[:, a:a + 1] + t * d[:, a:a + 1]
        ca = jnp.clip((pa / RADIUS + 1.0) * 0.5 * (RES - 1), 0.0, RES - 1)
        fa = jnp.floor(ca)
        cs.append(ca)
        fs.append(fa)
        ws.append(ca - fa)
    x0 = jnp.clip(fs[0].astype(jnp.int32), 0, RES - 1)
    y0 = jnp.clip(fs[1].astype(jnp.int32), 0, RES - 1)
    z0 = jnp.clip(fs[2].astype(jnp.int32), 0, RES - 1)
    x1 = jnp.minimum(x0 + 1, RES - 1)
    y1 = jnp.minimum(y0 + 1, RES - 1)
    z1 = jnp.minimum(z0 + 1, RES - 1)
    wx, wy, wz = ws
    for k in range(8):
        dz, dy, dx = (k >> 2) & 1, (k >> 1) & 1, k & 1
        zi = z1 if dz else z0
        yi = y1 if dy else y0
        xi = x1 if dx else x0
        flat = (zi * RES + yi) * RES + xi
        idx_ref[0, k] = jnp.clip(flat, 0, RES ** 3 - 1)
        wk = ((wz if dz else 1.0 - wz) * (wy if dy else 1.0 - wy)
              * (wx if dx else 1.0 - wx))
        w_ref[0, k] = wk


def _idxw_call(rays_o, rays_d):
    return pl.pallas_call(
        _idxw_body,
        grid=(NBLK,),
        in_specs=[
            pl.BlockSpec((RB, 3), lambda i: (i, 0)),
            pl.BlockSpec((RB, 3), lambda i: (i, 0)),
        ],
        out_specs=[
            pl.BlockSpec((1, 8, RB, NPAD), lambda i: (i, 0, 0, 0)),
            pl.BlockSpec((1, 8, RB, NPAD), lambda i: (i, 0, 0, 0)),
        ],
        out_shape=[
            jax.ShapeDtypeStruct((NBLK, 8, RB, NPAD), jnp.int32),
            jax.ShapeDtypeStruct((NBLK, 8, RB, NPAD), jnp.float32),
        ],
    )(rays_o, rays_d)


def _sc_gather_body(table_hbm, idx_hbm, w_hbm, out0_hbm, out1_hbm, idxb, wb,
                    rows, acc, sem_a, sem_b, sem_o):
    core = lax.axis_index("c")
    sub = lax.axis_index("s")
    rb = core * (NBLK // SC_CORES) + sub // 2
    half = sub % 2
    sems = (sem_a, sem_b)
    ntiles = ROWS_PER_CHUNK * QT

    def fire(tt, buf):
        r = tt // QT
        q = tt % QT
        return [
            pltpu.async_copy(
                table_hbm.at[idxb.at[k, r, pl.ds(q * 128, 128)]],
                rows.at[buf, k], sems[buf])
            for k in range(8)
        ]

    def chunk_body(c, _):
        row0 = half * ROWS_PER_WORKER + c * ROWS_PER_CHUNK
        pltpu.sync_copy(idx_hbm.at[rb, :, pl.ds(row0, ROWS_PER_CHUNK), :],
                        idxb)
        pltpu.sync_copy(w_hbm.at[rb, :, pl.ds(row0, ROWS_PER_CHUNK), :], wb)
        out_pending = 0
        pending = fire(0, 0)
        for tt in range(ntiles):
            buf = tt % 2
            nxt = fire(tt + 1, 1 - buf) if tt + 1 < ntiles else None
            for cp in pending:
                cp.wait()
            if out_pending >= 2:
                pltpu.make_async_copy(
                    out0_hbm.at[pl.ds(0, 128), :], acc.at[buf],
                    sem_o).wait()
                out_pending -= 1
            r = tt // QT
            q = tt % QT

            def group_body(g, _, buf=buf, r=r, q=q):
                j0 = g * 16
                wvs = [wb[k, r, pl.ds(q * 128 + j0, 16)] for k in range(8)]
                for jj in range(16):
                    j = j0 + jj
                    s0 = jnp.zeros((16,), jnp.float32)
                    s1 = jnp.zeros((16,), jnp.float32)
                    for k in range(8):
                        wk = wvs[k][jj]
                        s0 = s0 + rows[buf, k, j, 0:16] * wk
                        s1 = s1 + rows[buf, k, j, 16:32] * wk
                    acc[buf, j, 0:16] = s0
                    acc[buf, j, 16:32] = s1
                return 0

            lax.fori_loop(0, 8, group_body, 0)
            base = (((sub // 2) * RB) + row0 + r) * NPAD + q * 128

            @pl.when(core == 0)
            def _():
                pltpu.async_copy(acc.at[buf],
                                 out0_hbm.at[pl.ds(base, 128), :], sem_o)

            @pl.when(core == 1)
            def _():
                pltpu.async_copy(acc.at[buf],
                                 out1_hbm.at[pl.ds(base, 128), :], sem_o)

            out_pending += 1
            if nxt is not None:
                pending = nxt
        for _ in range(out_pending):
            pltpu.make_async_copy(out0_hbm.at[pl.ds(0, 128), :],
                                  acc.at[0], sem_o).wait()
        return 0

    lax.fori_loop(0, NCHUNK, chunk_body, 0)


@functools.lru_cache(maxsize=1)
def _sc_gather_kernel():
    return pl.kernel(
        _sc_gather_body,
        out_type=[jax.ShapeDtypeStruct((NPTS // 2, PADC), jnp.float32),
                  jax.ShapeDtypeStruct((NPTS // 2, PADC), jnp.float32)],
        mesh=plsc.VectorSubcoreMesh(core_axis_name="c", subcore_axis_name="s",
                                    num_cores=SC_CORES,
                                    num_subcores=SC_SUBCORES),
        scratch_types=[
            pltpu.VMEM((8, ROWS_PER_CHUNK, NPAD), jnp.int32),
            pltpu.VMEM((8, ROWS_PER_CHUNK, NPAD), jnp.float32),
            pltpu.VMEM((2, 8, 128, PADC), jnp.float32),
            pltpu.VMEM((2, 128, PADC), jnp.float32),
            pltpu.SemaphoreType.DMA,
            pltpu.SemaphoreType.DMA,
            pltpu.SemaphoreType.DMA,
        ],
        compiler_params=pltpu.CompilerParams(use_tc_tiling_on_sc=False),
    )


CRB = 32


def _compose_body(o_ref, d_ref, interp_ref, out_ref):
    o = o_ref[...]
    d = d_ref[...]
    interp = interp_ref[0]
    start = _ray_start(o, d)
    dn = jnp.sqrt(jnp.sum(d * d, axis=-1, keepdims=True))
    dist = STEP * dn
    ni = lax.broadcasted_iota(jnp.int32, (CRB, NPAD), 1)
    t = start + ni.astype(jnp.float32) * STEP
    mask = ni < NS
    for a in range(3):
        pa = o[:, a:a + 1] + t * d[:, a:a + 1]
        mask = mask & (pa > -RADIUS) & (pa < RADIUS)
    x = d[:, 0:1]
    y = d[:, 1:2]
    z = d[:, 2:3]
    sh = jnp.concatenate([
        jnp.full_like(x, C0),
        -C1 * y, C1 * z, -C1 * x,
        C2[0] * x * y, C2[1] * y * z,
        C2[2] * (2.0 * z * z - x * x - y * y),
        C2[3] * x * z, C2[4] * (x * x - y * y),
    ], axis=-1)
    shmext = jnp.concatenate(
        [sh, sh, sh, jnp.ones((CRB, 1), jnp.float32),
         jnp.zeros((CRB, PADC - CH), jnp.float32)], axis=-1)
    u = (interp * shmext[:, None, :]).reshape(CRB * NPAD, PADC)
    chi = lax.broadcasted_iota(jnp.int32, (PADC, 8), 0)
    cci = lax.broadcasted_iota(jnp.int32, (PADC, 8), 1)
    sel = (((chi < 27) & (chi // 9 == cci)) | ((chi == 27) & (cci == 3)))
    out4 = jnp.dot(u, sel.astype(jnp.float32),
                   preferred_element_type=jnp.float32)
    out4 = out4.reshape(CRB, NPAD, 8)
    sigma = jnp.maximum(jnp.where(mask, out4[:, :, 3], 0.0), 0.0)
    sigd = sigma * dist
    alpha = 1.0 - jnp.exp(-sigd)
    mi = lax.broadcasted_iota(jnp.int32, (NPAD, NPAD), 0)
    nj = lax.broadcasted_iota(jnp.int32, (NPAD, NPAD), 1)
    lower = (mi < nj).astype(jnp.float32)
    cum = jnp.dot(sigd, lower, preferred_element_type=jnp.float32)
    trans = jnp.exp(-cum)
    wlight = alpha * trans
    bg = 1.0 - jnp.sum(wlight, axis=-1)
    cols = []
    for c in range(3):
        rgb_c = jax.nn.sigmoid(jnp.where(mask, out4[:, :, c], 0.0))
        cols.append(jnp.sum(wlight * rgb_c, axis=-1) + bg)
    out_ref[...] = jnp.stack(cols, axis=-1)


def _compose_call(rays_o, rays_d, interp4):
    nrays = rays_o.shape[0]
    return pl.pallas_call(
        _compose_body,
        grid=(nrays // CRB,),
        in_specs=[
            pl.BlockSpec((CRB, 3), lambda i: (i, 0)),
            pl.BlockSpec((CRB, 3), lambda i: (i, 0)),
            pl.BlockSpec((1, CRB, NPAD, PADC), lambda i: (i, 0, 0, 0)),
        ],
        out_specs=pl.BlockSpec((CRB, 3), lambda i: (i, 0)),
        out_shape=jax.ShapeDtypeStruct((nrays, 3), jnp.float32),
    )(rays_o, rays_d, interp4)


def kernel(rays_o, rays_d, data):
    table = jnp.pad(data.reshape(CH, RES ** 3).T,
                    ((0, 0), (0, PADC - CH)))
    idx, w = _idxw_call(rays_o, rays_d)
    interp0, interp1 = _sc_gather_kernel()(table, idx, w)
    halves = []
    for h, interp in enumerate((interp0, interp1)):
        sl = slice(h * 512, (h + 1) * 512)
        interp4 = interp.reshape(512 // CRB, CRB, NPAD, PADC)
        halves.append(_compose_call(rays_o[sl], rays_d[sl], interp4))
    return jnp.concatenate(halves, axis=0)
